# baseline (speedup 1.0000x reference)
import functools

import jax
import jax.numpy as jnp
from jax import lax
from jax.experimental import pallas as pl
from jax.experimental.pallas import tpu as pltpu
from jax.experimental.pallas import tpu_sc as plsc

NUM_CORES = 2
NUM_SUBCORES = 16
NW = NUM_CORES * NUM_SUBCORES

CB = 256
SUB = CB // 128
NBUF = 2


@functools.partial(jax.jit, static_argnames=())
def _gather_t(idxT, weight):
    H, Bt = idxT.shape
    D = weight.shape[1]
    DB = D // 8
    bpw = Bt // NW
    mesh = plsc.VectorSubcoreMesh(core_axis_name="c", subcore_axis_name="s")

    @functools.partial(
        pl.kernel,
        mesh=mesh,
        out_type=jax.ShapeDtypeStruct((H, DB, (Bt // 128) * 8, 128),
                                      jnp.float32),
        scratch_types=[
            pltpu.VMEM((NBUF, CB), jnp.int32),
            pltpu.VMEM((NBUF, CB, D), jnp.float32),
            pltpu.VMEM((NBUF, DB * SUB * 8, 129), jnp.float32),
            pltpu.SemaphoreType.DMA,
            pltpu.SemaphoreType.DMA,
            pltpu.SemaphoreType.DMA,
            pltpu.SemaphoreType.DMA,
        ],
        compiler_params=pltpu.CompilerParams(use_tc_tiling_on_sc=False,
                                             needs_layout_passes=False),
    )
    def k(idx_hbm, table_hbm, out_hbm, idx_v, rows_v, tbuf, g0, g1, o0, o1):
        gsems = (g0, g1)
        osems = (o0, o1)
        wid = lax.axis_index("s") * NUM_CORES + lax.axis_index("c")
        b0w = wid * bpw
        iota = lax.iota(jnp.int32, 16)
        rvs = [((iota + d0) // 8 * SUB + bo) * 8 + (iota + d0) % 8
               for bo in range(SUB) for d0 in (0, 16, 32, 48)]

        def group(h, carry):
            descs = []
            for b in range(NBUF):
                blk0 = wid * (bpw // 128) + b * SUB

                @pl.when(h > 0)
                def _drain():
                    for db in range(DB):
                        pltpu.make_async_copy(
                            tbuf.at[b, pl.ds(db * SUB * 8, SUB * 8),
                                    pl.ds(0, 128)],
                            out_hbm.at[h - 1, db, pl.ds(blk0 * 8, SUB * 8)],
                            osems[b],
                        ).wait()

                pltpu.sync_copy(
                    idx_hbm.at[h, pl.ds(b0w + b * CB, CB)], idx_v.at[b])
                descs.append(
                    pltpu.async_copy(table_hbm.at[idx_v.at[b]], rows_v.at[b],
                                     gsems[b]))
            for b in range(NBUF):
                blk0 = wid * (bpw // 128) + b * SUB
                descs[b].wait()
                rows2 = rows_v.at[b]
                tb = tbuf.at[b]
                for bo in range(SUB):
                    rv = rvs[bo * 4:bo * 4 + 4]

                    @plsc.parallel_loop(0, 128, 1, unroll=8)
                    def _t_body(bi, bo=bo, rv=rv):
                        cvec = jnp.zeros((16,), jnp.int32) + bi
                        for d0g in range(4):
                            v = rows2[bo * 128 + bi, pl.ds(d0g * 16, 16)]
                            plsc.store_scatter(tb, [rv[d0g], cvec], v)
                for db in range(DB):
                    pltpu.async_copy(
                        tb.at[pl.ds(db * SUB * 8, SUB * 8), pl.ds(0, 128)],
                        out_hbm.at[h, db, pl.ds(blk0 * 8, SUB * 8)],
                        osems[b])
            return carry

        lax.fori_loop(0, H, group, 0)
        for b in range(NBUF):
            blk0 = wid * (bpw // 128) + b * SUB
            for db in range(DB):
                pltpu.make_async_copy(
                    tbuf.at[b, pl.ds(db * SUB * 8, SUB * 8), pl.ds(0, 128)],
                    out_hbm.at[H - 1, db, pl.ds(blk0 * 8, SUB * 8)],
                    osems[b]).wait()

    return k(idxT, weight)


def kernel(input_, weight):
    Bt, H = input_.shape
    D = weight.shape[1]
    idxT = jnp.transpose(input_).astype(jnp.int32)
    out4 = _gather_t(idxT, weight)
    out5 = out4.reshape(H, 8, Bt // 128, 8, 128)
    return out5.transpose(2, 4, 0, 1, 3).reshape(Bt, H, D)


# speedup vs baseline: 1.0053x; 1.0053x over previous
import functools

import jax
import jax.numpy as jnp
from jax import lax
from jax.experimental import pallas as pl
from jax.experimental.pallas import tpu as pltpu
from jax.experimental.pallas import tpu_sc as plsc

NUM_CORES = 2
NUM_SUBCORES = 16
NW = NUM_CORES * NUM_SUBCORES

CB = 256
SUB = CB // 128
NBUF = 2


@functools.partial(jax.jit, static_argnames=())
def _gather_t(idxT, weight):
    H, Bt = idxT.shape
    D = weight.shape[1]
    DB = D // 8
    bpw = Bt // NW
    mesh = plsc.VectorSubcoreMesh(core_axis_name="c", subcore_axis_name="s")

    @functools.partial(
        pl.kernel,
        mesh=mesh,
        out_type=jax.ShapeDtypeStruct((H, DB, (Bt // 128) * 8, 128),
                                      jnp.float32),
        scratch_types=[
            pltpu.VMEM((NBUF, CB), jnp.int32),
            pltpu.VMEM((NBUF, CB, D), jnp.float32),
            pltpu.VMEM((NBUF, DB, SUB * 8, 129), jnp.float32),
            pltpu.SemaphoreType.DMA,
            pltpu.SemaphoreType.DMA,
            pltpu.SemaphoreType.DMA,
            pltpu.SemaphoreType.DMA,
        ],
        compiler_params=pltpu.CompilerParams(use_tc_tiling_on_sc=False,
                                             needs_layout_passes=False),
    )
    def k(idx_hbm, table_hbm, out_hbm, idx_v, rows_v, tbuf, g0, g1, o0, o1):
        gsems = (g0, g1)
        osems = (o0, o1)
        wid = lax.axis_index("s") * NUM_CORES + lax.axis_index("c")
        b0w = wid * bpw
        iota = lax.iota(jnp.int32, 16)
        dbs = [(iota + d0) // 8 for d0 in (0, 16, 32, 48)]
        rvs = [bo * 8 + (iota + d0) % 8
               for bo in range(SUB) for d0 in (0, 16, 32, 48)]

        def group(h, carry):
            descs = []
            for b in range(NBUF):
                blk0 = wid * (bpw // 128) + b * SUB

                @pl.when(h > 0)
                def _drain():
                    pltpu.make_async_copy(
                        tbuf.at[b, :, :, pl.ds(0, 128)],
                        out_hbm.at[h - 1, :, pl.ds(blk0 * 8, SUB * 8)],
                        osems[b],
                    ).wait()

                pltpu.sync_copy(
                    idx_hbm.at[h, pl.ds(b0w + b * CB, CB)], idx_v.at[b])
                descs.append(
                    pltpu.async_copy(table_hbm.at[idx_v.at[b]], rows_v.at[b],
                                     gsems[b]))
            for b in range(NBUF):
                blk0 = wid * (bpw // 128) + b * SUB
                descs[b].wait()
                rows2 = rows_v.at[b]
                tb = tbuf.at[b]
                for bo in range(SUB):
                    rv = rvs[bo * 4:bo * 4 + 4]

                    @plsc.parallel_loop(0, 128, 1, unroll=4)
                    def _t_body(bi, bo=bo, rv=rv):
                        cvec = jnp.zeros((16,), jnp.int32) + bi
                        for d0g in range(4):
                            v = rows2[bo * 128 + bi, pl.ds(d0g * 16, 16)]
                            plsc.store_scatter(tb, [dbs[d0g], rv[d0g], cvec],
                                               v)
                pltpu.async_copy(
                    tb.at[:, :, pl.ds(0, 128)],
                    out_hbm.at[h, :, pl.ds(blk0 * 8, SUB * 8)],
                    osems[b])
            return carry

        lax.fori_loop(0, H, group, 0)
        for b in range(NBUF):
            blk0 = wid * (bpw // 128) + b * SUB
            pltpu.make_async_copy(
                tbuf.at[b, :, :, pl.ds(0, 128)],
                out_hbm.at[H - 1, :, pl.ds(blk0 * 8, SUB * 8)],
                osems[b]).wait()

    return k(idxT, weight)


def kernel(input_, weight):
    Bt, H = input_.shape
    D = weight.shape[1]
    idxT = jnp.transpose(input_).astype(jnp.int32)
    out4 = _gather_t(idxT, weight)
    out5 = out4.reshape(H, 8, Bt // 128, 8, 128)
    return out5.transpose(2, 4, 0, 1, 3).reshape(Bt, H, D)
